# baseline (speedup 1.0000x reference)
import functools

import jax
import jax.numpy as jnp
from jax import lax
from jax.experimental import pallas as pl
from jax.experimental.pallas import tpu as pltpu
from jax.experimental.pallas import tpu_sc as plsc

_N2 = 10000
_N3 = 2048
_D = 128
_DO = 47
_W1 = 144
_W2 = 64
_NC, _NS = 2, 16
_NW = _NC * _NS
_CH = 128
_NB = 4
_C1 = 40
_C2 = 8
_ACC1_ROWS = _N3 + 128


def _mesh():
    return plsc.VectorSubcoreMesh(core_axis_name="c", subcore_axis_name="s",
                                  num_cores=_NC, num_subcores=_NS)


def _a_body(x_ref, w_ref, c_ref, y_ref):
    y_ref[...] = jnp.dot(x_ref[...], w_ref[...],
                         preferred_element_type=jnp.float32) + c_ref[...]


def _run_a(x10k, w1l_ext, cnt_row):
    blk = 1000
    grid = _N2 // blk
    return pl.pallas_call(
        _a_body,
        grid=(grid,),
        in_specs=[
            pl.BlockSpec((blk, _D), lambda i: (i, 0)),
            pl.BlockSpec((_D, _W1), lambda i: (0, 0)),
            pl.BlockSpec((1, _W1), lambda i: (0, 0)),
        ],
        out_specs=pl.BlockSpec((blk, _W1), lambda i: (i, 0)),
        out_shape=jax.ShapeDtypeStruct((_N2, _W1), jnp.float32),
    )(x10k, w1l_ext, cnt_row)


def _sc_body(nchunks, acc_rows, width, tab_hbm, src_hbm, dst_hbm, zeros_hbm,
             out_hbm, src_v, dst_v, rows_v, acc_sh, gsem, ssem):
    cid = lax.axis_index("c")
    sid = lax.axis_index("s")
    wid = sid * _NC + cid
    zrows = acc_rows // _NS
    orows = _N3 // _NS
    pltpu.sync_copy(zeros_hbm, acc_sh.at[pl.ds(sid * zrows, zrows)])
    pltpu.sync_copy(src_hbm.at[wid], src_v)
    pltpu.sync_copy(dst_hbm.at[wid], dst_v)
    plsc.subcore_barrier()

    @pl.loop(0, nchunks, step=_NB)
    def _group(g):
        gds = [pltpu.async_copy(tab_hbm.at[src_v.at[g + b]], rows_v.at[b],
                                gsem.at[b]) for b in range(_NB)]
        sds = []
        for b in range(_NB):
            gds[b].wait()
            sds.append(pltpu.async_copy(rows_v.at[b], acc_sh.at[dst_v.at[g + b]],
                                        ssem.at[b], add=True))
        for d in sds:
            d.wait()

    plsc.subcore_barrier()
    pltpu.sync_copy(acc_sh.at[pl.ds(sid * orows, orows)],
                    out_hbm.at[cid, pl.ds(sid * orows, orows)])


def _run_sc(tab, src, dst, zeros, nchunks, acc_rows, width):
    kern = functools.partial(
        pl.kernel,
        out_type=jax.ShapeDtypeStruct((_NC, _N3, width), jnp.float32),
        mesh=_mesh(),
        scratch_types=[
            pltpu.VMEM((nchunks, _CH), jnp.int32),
            pltpu.VMEM((nchunks, _CH), jnp.int32),
            pltpu.VMEM((_NB, _CH, width), jnp.float32),
            pltpu.VMEM_SHARED((acc_rows, width), jnp.float32),
            pltpu.SemaphoreType.DMA((_NB,)),
            pltpu.SemaphoreType.DMA((_NB,)),
        ],
        compiler_params=pltpu.CompilerParams(use_tc_tiling_on_sc=False),
    )(functools.partial(_sc_body, nchunks, acc_rows, width))
    return kern(tab, src, dst, zeros)


_CAP = 163840 // _NW
_PK = 16384
_ST = 8


def _sc1_body(tab_hbm, pack_hbm, zsrc_hbm, d2d_hbm, zeros_hbm,
              out_hbm, pinp, csrc2d, cdst2d, rows_v, acc_sh, gsem, ssem):
    cid = lax.axis_index("c")
    sid = lax.axis_index("s")
    wid = sid * _NC + cid
    zrows = _ACC1_ROWS // _NS
    orows = _N3 // _NS
    pltpu.sync_copy(zeros_hbm, acc_sh.at[pl.ds(sid * zrows, zrows)])
    pltpu.sync_copy(pack_hbm.at[wid], pinp)
    pltpu.sync_copy(zsrc_hbm, csrc2d)
    pltpu.sync_copy(d2d_hbm, cdst2d)

    def _p1(i, off):
        p = pinp[pl.ds(i * 16, 16)]
        m = p < (_N3 * _PK)
        mi = m.astype(jnp.int32)
        cs = plsc.cumsum(mi)
        pos = off + cs - 1
        rowv = lax.shift_right_logical(pos, 7)
        colv = jnp.bitwise_and(pos, _CH - 1)
        plsc.store_scatter(csrc2d, [rowv, colv], jnp.bitwise_and(p, _PK - 1),
                           mask=m)
        plsc.store_scatter(cdst2d, [rowv, colv], lax.shift_right_logical(p, 14),
                           mask=m)
        return off + jnp.sum(mi)

    off = lax.fori_loop(0, _CAP // 16, _p1, jnp.int32(0))
    plsc.subcore_barrier()

    for g in range(_ST // _NB):
        c0 = g * _NB
        gds = [pltpu.async_copy(tab_hbm.at[csrc2d.at[c0 + b]], rows_v.at[b],
                                gsem.at[b]) for b in range(_NB)]
        sds = []
        for b in range(_NB):
            gds[b].wait()
            sds.append(pltpu.async_copy(rows_v.at[b],
                                        acc_sh.at[cdst2d.at[c0 + b]],
                                        ssem.at[b], add=True))
        for d in sds:
            d.wait()

    @pl.loop(_ST, (off + _CH - 1) // _CH)
    def _p3(c):
        pltpu.sync_copy(tab_hbm.at[csrc2d.at[c]], rows_v.at[0])
        pltpu.sync_copy(rows_v.at[0], acc_sh.at[cdst2d.at[c]], add=True)
    plsc.subcore_barrier()
    pltpu.sync_copy(acc_sh.at[pl.ds(sid * orows, orows)],
                    out_hbm.at[cid, pl.ds(sid * orows, orows)])


def _run_sc1(tab, packed, zsrc, d2d, zeros):
    kern = functools.partial(
        pl.kernel,
        out_type=jax.ShapeDtypeStruct((_NC, _N3, _W1), jnp.float32),
        mesh=_mesh(),
        scratch_types=[
            pltpu.VMEM((_CAP,), jnp.int32),
            pltpu.VMEM((_CAP // _CH, _CH), jnp.int32),
            pltpu.VMEM((_CAP // _CH, _CH), jnp.int32),
            pltpu.VMEM((_NB, _CH, _W1), jnp.float32),
            pltpu.VMEM_SHARED((_ACC1_ROWS, _W1), jnp.float32),
            pltpu.SemaphoreType.DMA((_NB,)),
            pltpu.SemaphoreType.DMA((_NB,)),
        ],
        compiler_params=pltpu.CompilerParams(use_tc_tiling_on_sc=False,
                                             needs_layout_passes=False),
    )(_sc1_body)
    return kern(tab, packed, zsrc, d2d, zeros)


def _c_body(p_ref, x2_ref, w1r_ref, b1_ref, w2le_ref, c2_ref, w2re_ref, b2_ref,
            y2_ref, z2_ref):
    s = p_ref[0] + p_ref[1]
    cnt = jnp.maximum(s[:, _D:_D + 1], 1.0)
    mean = s[:, :_D] / cnt
    h = mean + jnp.dot(x2_ref[...], w1r_ref[...],
                       preferred_element_type=jnp.float32) + b1_ref[...]
    h = jnp.maximum(h, 0.0)
    y2_ref[...] = jnp.dot(h, w2le_ref[...],
                          preferred_element_type=jnp.float32) + c2_ref[...]
    z2_ref[...] = jnp.dot(h, w2re_ref[...],
                          preferred_element_type=jnp.float32) + b2_ref[...]


def _run_c(p1, x2, w1r, b1, w2l_ext, c2_row, w2r_ext, b2_ext):
    return pl.pallas_call(
        _c_body,
        out_shape=(jax.ShapeDtypeStruct((_N3, _W2), jnp.float32),
                   jax.ShapeDtypeStruct((_N3, _W2), jnp.float32)),
    )(p1, x2, w1r, b1, w2l_ext, c2_row, w2r_ext, b2_ext)


def _e_body(p_ref, z2_ref, out_ref):
    s = p_ref[0] + p_ref[1]
    cnt = jnp.maximum(s[:, 48:49], 1.0)
    logits = s / cnt + z2_ref[...]
    col = lax.broadcasted_iota(jnp.int32, (_N3, _W2), 1)
    ml = jnp.where(col < _DO, logits, -1e30)
    m = jnp.max(ml, axis=1, keepdims=True)
    lse = jnp.log(jnp.sum(jnp.exp(ml - m), axis=1, keepdims=True)) + m
    out_ref[...] = (ml - lse)[:, :_DO]


def _run_e(p2, z2b):
    return pl.pallas_call(
        _e_body,
        out_shape=jax.ShapeDtypeStruct((_N3, _DO), jnp.float32),
    )(p2, z2b)


def kernel(x, edge_index1, edge_index2, W1l, b1, W1r, W2l, b2, W2r):
    f32 = jnp.float32
    x10k = x[:_N2]
    x2 = x[:_N3]

    w1l_ext = jnp.zeros((_D, _W1), f32).at[:, :_D].set(W1l)
    cnt_row = jnp.zeros((1, _W1), f32).at[0, _D].set(1.0)
    w2l_ext = jnp.zeros((_D, _W2), f32).at[:, :_DO].set(W2l)
    c2_row = jnp.zeros((1, _W2), f32).at[0, 48].set(1.0)
    w2r_ext = jnp.zeros((_D, _W2), f32).at[:, :_DO].set(W2r)
    b2_ext = jnp.zeros((1, _W2), f32).at[0, :_DO].set(b2)
    b1r = b1.reshape(1, _D)

    e1p = _NW * _CAP
    src1 = edge_index1[0].astype(jnp.int32)
    dst1 = edge_index1[1].astype(jnp.int32)
    pad = e1p - src1.shape[0]
    packed = dst1 * _PK + src1
    packed = jnp.concatenate([packed, jnp.full((pad,), _N2 * _PK, jnp.int32)])
    packed = packed.reshape(_NW, _CAP)
    zsrc = jnp.zeros((_CAP // _CH, _CH), jnp.int32)
    d2d = (_N3 + (jnp.arange(_CAP, dtype=jnp.int32) & 127)).reshape(
        _CAP // _CH, _CH)
    src2 = edge_index2[0].astype(jnp.int32).reshape(_NW, _C2, _CH)
    dst2 = edge_index2[1].astype(jnp.int32).reshape(_NW, _C2, _CH)

    zeros1 = jnp.zeros((_ACC1_ROWS // _NS, _W1), f32)
    zeros2 = jnp.zeros((_N3 // _NS, _W2), f32)

    y1ext = _run_a(x10k, w1l_ext, cnt_row)
    p1 = _run_sc1(y1ext, packed, zsrc, d2d, zeros1)
    y2ext, z2b = _run_c(p1, x2, W1r, b1r, w2l_ext, c2_row, w2r_ext, b2_ext)
    p2 = _run_sc(y2ext, src2, dst2, zeros2, _C2, _N3, _W2)
    return _run_e(p2, z2b)


# speedup vs baseline: 1.3286x; 1.3286x over previous
import functools

import jax
import jax.numpy as jnp
from jax import lax
from jax.experimental import pallas as pl
from jax.experimental.pallas import tpu as pltpu
from jax.experimental.pallas import tpu_sc as plsc

_N2 = 10000
_N3 = 2048
_D = 128
_DO = 47
_W1 = 160
_W2 = 64
_NC, _NS = 2, 16
_NW = _NC * _NS
_CH = 128
_NB = 4
_C1 = 40
_C2 = 8
_ACC1_ROWS = _N3 + 128


def _mesh():
    return plsc.VectorSubcoreMesh(core_axis_name="c", subcore_axis_name="s",
                                  num_cores=_NC, num_subcores=_NS)


def _a_body(x_ref, w_ref, c_ref, y_ref):
    y_ref[...] = (jnp.dot(x_ref[...], w_ref[...],
                          preferred_element_type=jnp.float32)
                  + c_ref[...]).astype(jnp.bfloat16)


def _run_a(x10k, w1l_ext, cnt_row):
    blk = 1000
    grid = _N2 // blk
    return pl.pallas_call(
        _a_body,
        grid=(grid,),
        in_specs=[
            pl.BlockSpec((blk, _D), lambda i: (i, 0)),
            pl.BlockSpec((_D, _W1), lambda i: (0, 0)),
            pl.BlockSpec((1, _W1), lambda i: (0, 0)),
        ],
        out_specs=pl.BlockSpec((blk, _W1), lambda i: (i, 0)),
        out_shape=jax.ShapeDtypeStruct((_N2, _W1), jnp.bfloat16),
    )(x10k, w1l_ext, cnt_row)


def _sc_body(nchunks, acc_rows, width, tab_hbm, src_hbm, dst_hbm, zeros_hbm,
             out_hbm, src_v, dst_v, rows_v, acc_sh, gsem, ssem):
    cid = lax.axis_index("c")
    sid = lax.axis_index("s")
    wid = sid * _NC + cid
    zrows = acc_rows // _NS
    orows = _N3 // _NS
    pltpu.sync_copy(zeros_hbm, acc_sh.at[pl.ds(sid * zrows, zrows)])
    pltpu.sync_copy(src_hbm.at[wid], src_v)
    pltpu.sync_copy(dst_hbm.at[wid], dst_v)
    plsc.subcore_barrier()

    @pl.loop(0, nchunks, step=_NB)
    def _group(g):
        gds = [pltpu.async_copy(tab_hbm.at[src_v.at[g + b]], rows_v.at[b],
                                gsem.at[b]) for b in range(_NB)]
        sds = []
        for b in range(_NB):
            gds[b].wait()
            sds.append(pltpu.async_copy(rows_v.at[b], acc_sh.at[dst_v.at[g + b]],
                                        ssem.at[b], add=True))
        for d in sds:
            d.wait()

    plsc.subcore_barrier()
    pltpu.sync_copy(acc_sh.at[pl.ds(sid * orows, orows)],
                    out_hbm.at[cid, pl.ds(sid * orows, orows)])


def _run_sc(tab, src, dst, zeros, nchunks, acc_rows, width):
    kern = functools.partial(
        pl.kernel,
        out_type=jax.ShapeDtypeStruct((_NC, _N3, width), jnp.float32),
        mesh=_mesh(),
        scratch_types=[
            pltpu.VMEM((nchunks, _CH), jnp.int32),
            pltpu.VMEM((nchunks, _CH), jnp.int32),
            pltpu.VMEM((_NB, _CH, width), jnp.float32),
            pltpu.VMEM_SHARED((acc_rows, width), jnp.float32),
            pltpu.SemaphoreType.DMA((_NB,)),
            pltpu.SemaphoreType.DMA((_NB,)),
        ],
        compiler_params=pltpu.CompilerParams(use_tc_tiling_on_sc=False),
    )(functools.partial(_sc_body, nchunks, acc_rows, width))
    return kern(tab, src, dst, zeros)


_CAP = 163840 // _NW
_PK = 16384
_ST = 8


def _sc1_body(tab_hbm, pack_hbm, zsrc_hbm, d2d_hbm, zeros_hbm,
              out_hbm, pinp, csrc2d, cdst2d, rows_v, acc_sh, gsem, ssem):
    cid = lax.axis_index("c")
    sid = lax.axis_index("s")
    wid = sid * _NC + cid
    zrows = _ACC1_ROWS // _NS
    orows = _N3 // _NS
    pltpu.sync_copy(zeros_hbm, acc_sh.at[pl.ds(sid * zrows, zrows)])
    pltpu.sync_copy(pack_hbm.at[wid], pinp)
    pltpu.sync_copy(zsrc_hbm, csrc2d)
    pltpu.sync_copy(d2d_hbm, cdst2d)

    def _p1(i, off):
        p = pinp[pl.ds(i * 16, 16)]
        m = p < (_N3 * _PK)
        mi = m.astype(jnp.int32)
        cs = plsc.cumsum(mi)
        pos = off + cs - 1
        rowv = lax.shift_right_logical(pos, 7)
        colv = jnp.bitwise_and(pos, _CH - 1)
        plsc.store_scatter(csrc2d, [rowv, colv], jnp.bitwise_and(p, _PK - 1),
                           mask=m)
        plsc.store_scatter(cdst2d, [rowv, colv], lax.shift_right_logical(p, 14),
                           mask=m)
        return off + jnp.sum(mi)

    off = lax.fori_loop(0, _CAP // 16, _p1, jnp.int32(0))
    plsc.subcore_barrier()

    @pl.loop(0, (off + _CH - 1) // _CH)
    def _p3(c):
        pltpu.sync_copy(tab_hbm.at[csrc2d.at[c]], rows_v.at[0])
        pltpu.sync_copy(rows_v.at[0], acc_sh.at[cdst2d.at[c]], add=True)
    plsc.subcore_barrier()
    pltpu.sync_copy(acc_sh.at[pl.ds(sid * orows, orows)],
                    out_hbm.at[cid, pl.ds(sid * orows, orows)])


def _run_sc1(tab, packed, zsrc, d2d, zeros):
    kern = functools.partial(
        pl.kernel,
        out_type=jax.ShapeDtypeStruct((_NC, _N3, _W1), jnp.bfloat16),
        mesh=_mesh(),
        scratch_types=[
            pltpu.VMEM((_CAP,), jnp.int32),
            pltpu.VMEM((_CAP // _CH, _CH), jnp.int32),
            pltpu.VMEM((_CAP // _CH, _CH), jnp.int32),
            pltpu.VMEM((_NB, _CH, _W1), jnp.bfloat16),
            pltpu.VMEM_SHARED((_ACC1_ROWS, _W1), jnp.bfloat16),
            pltpu.SemaphoreType.DMA((_NB,)),
            pltpu.SemaphoreType.DMA((_NB,)),
        ],
        compiler_params=pltpu.CompilerParams(use_tc_tiling_on_sc=False,
                                             needs_layout_passes=False),
    )(_sc1_body)
    return kern(tab, packed, zsrc, d2d, zeros)


def _c_body(p_ref, x2_ref, w1r_ref, b1_ref, w2le_ref, c2_ref, w2re_ref, b2_ref,
            y2_ref, z2_ref):
    s = (p_ref[0].astype(jnp.float32) + p_ref[1].astype(jnp.float32))
    cnt = jnp.maximum(s[:, _D:_D + 1], 1.0)
    mean = s[:, :_D] / cnt
    h = mean + jnp.dot(x2_ref[...], w1r_ref[...],
                       preferred_element_type=jnp.float32) + b1_ref[...]
    h = jnp.maximum(h, 0.0)
    y2_ref[...] = jnp.dot(h, w2le_ref[...],
                          preferred_element_type=jnp.float32) + c2_ref[...]
    z2_ref[...] = jnp.dot(h, w2re_ref[...],
                          preferred_element_type=jnp.float32) + b2_ref[...]


def _run_c(p1, x2, w1r, b1, w2l_ext, c2_row, w2r_ext, b2_ext):
    return pl.pallas_call(
        _c_body,
        out_shape=(jax.ShapeDtypeStruct((_N3, _W2), jnp.float32),
                   jax.ShapeDtypeStruct((_N3, _W2), jnp.float32)),
    )(p1, x2, w1r, b1, w2l_ext, c2_row, w2r_ext, b2_ext)


def _e_body(p_ref, z2_ref, out_ref):
    s = p_ref[0] + p_ref[1]
    cnt = jnp.maximum(s[:, 48:49], 1.0)
    logits = s / cnt + z2_ref[...]
    col = lax.broadcasted_iota(jnp.int32, (_N3, _W2), 1)
    ml = jnp.where(col < _DO, logits, -1e30)
    m = jnp.max(ml, axis=1, keepdims=True)
    lse = jnp.log(jnp.sum(jnp.exp(ml - m), axis=1, keepdims=True)) + m
    out_ref[...] = (ml - lse)[:, :_DO]


def _run_e(p2, z2b):
    return pl.pallas_call(
        _e_body,
        out_shape=jax.ShapeDtypeStruct((_N3, _DO), jnp.float32),
    )(p2, z2b)


def kernel(x, edge_index1, edge_index2, W1l, b1, W1r, W2l, b2, W2r):
    f32 = jnp.float32
    x10k = x[:_N2]
    x2 = x[:_N3]

    w1l_ext = jnp.zeros((_D, _W1), f32).at[:, :_D].set(W1l)
    cnt_row = jnp.zeros((1, _W1), f32).at[0, _D].set(1.0)
    w2l_ext = jnp.zeros((_D, _W2), f32).at[:, :_DO].set(W2l)
    c2_row = jnp.zeros((1, _W2), f32).at[0, 48].set(1.0)
    w2r_ext = jnp.zeros((_D, _W2), f32).at[:, :_DO].set(W2r)
    b2_ext = jnp.zeros((1, _W2), f32).at[0, :_DO].set(b2)
    b1r = b1.reshape(1, _D)

    e1p = _NW * _CAP
    src1 = edge_index1[0].astype(jnp.int32)
    dst1 = edge_index1[1].astype(jnp.int32)
    pad = e1p - src1.shape[0]
    packed = dst1 * _PK + src1
    packed = jnp.concatenate([packed, jnp.full((pad,), _N2 * _PK, jnp.int32)])
    packed = packed.reshape(_NW, _CAP)
    zsrc = jnp.zeros((_CAP // _CH, _CH), jnp.int32)
    d2d = (_N3 + (jnp.arange(_CAP, dtype=jnp.int32) & 127)).reshape(
        _CAP // _CH, _CH)
    src2 = edge_index2[0].astype(jnp.int32).reshape(_NW, _C2, _CH)
    dst2 = edge_index2[1].astype(jnp.int32).reshape(_NW, _C2, _CH)

    zeros1 = jnp.zeros((_ACC1_ROWS // _NS, _W1), jnp.bfloat16)
    zeros2 = jnp.zeros((_N3 // _NS, _W2), f32)

    y1ext = _run_a(x10k, w1l_ext, cnt_row)
    p1 = _run_sc1(y1ext, packed, zsrc, d2d, zeros1)
    y2ext, z2b = _run_c(p1, x2, W1r, b1r, w2l_ext, c2_row, w2r_ext, b2_ext)
    p2 = _run_sc(y2ext, src2, dst2, zeros2, _C2, _N3, _W2)
    return _run_e(p2, z2b)


# speedup vs baseline: 1.3604x; 1.0239x over previous
import functools

import jax
import jax.numpy as jnp
from jax import lax
from jax.experimental import pallas as pl
from jax.experimental.pallas import tpu as pltpu
from jax.experimental.pallas import tpu_sc as plsc

_N2 = 10000
_N3 = 2048
_D = 128
_DO = 47
_W1 = 160
_W2 = 64
_NC, _NS = 2, 16
_NW = _NC * _NS
_CH = 128
_NB = 4
_C1 = 40
_C2 = 8
_ACC1_ROWS = _N3 + 128


def _mesh():
    return plsc.VectorSubcoreMesh(core_axis_name="c", subcore_axis_name="s",
                                  num_cores=_NC, num_subcores=_NS)


def _a_body(x_ref, w_ref, c_ref, y_ref):
    y_ref[...] = (jnp.dot(x_ref[...], w_ref[...],
                          preferred_element_type=jnp.float32)
                  + c_ref[...]).astype(jnp.bfloat16)


def _run_a(x10k, w1l_ext, cnt_row):
    blk = 1000
    grid = _N2 // blk
    return pl.pallas_call(
        _a_body,
        grid=(grid,),
        in_specs=[
            pl.BlockSpec((blk, _D), lambda i: (i, 0)),
            pl.BlockSpec((_D, _W1), lambda i: (0, 0)),
            pl.BlockSpec((1, _W1), lambda i: (0, 0)),
        ],
        out_specs=pl.BlockSpec((blk, _W1), lambda i: (i, 0)),
        out_shape=jax.ShapeDtypeStruct((_N2, _W1), jnp.bfloat16),
    )(x10k, w1l_ext, cnt_row)


def _sc_body(nchunks, acc_rows, width, tab_hbm, src_hbm, dst_hbm, zeros_hbm,
             out_hbm, src_v, dst_v, rows_v, acc_sh, gsem, ssem):
    cid = lax.axis_index("c")
    sid = lax.axis_index("s")
    wid = sid * _NC + cid
    zrows = acc_rows // _NS
    orows = _N3 // _NS
    pltpu.sync_copy(zeros_hbm, acc_sh.at[pl.ds(sid * zrows, zrows)])
    pltpu.sync_copy(src_hbm.at[wid], src_v)
    pltpu.sync_copy(dst_hbm.at[wid], dst_v)
    plsc.subcore_barrier()

    @pl.loop(0, nchunks, step=_NB)
    def _group(g):
        gds = [pltpu.async_copy(tab_hbm.at[src_v.at[g + b]], rows_v.at[b],
                                gsem.at[b]) for b in range(_NB)]
        sds = []
        for b in range(_NB):
            gds[b].wait()
            sds.append(pltpu.async_copy(rows_v.at[b], acc_sh.at[dst_v.at[g + b]],
                                        ssem.at[b], add=True))
        for d in sds:
            d.wait()

    plsc.subcore_barrier()
    pltpu.sync_copy(acc_sh.at[pl.ds(sid * orows, orows)],
                    out_hbm.at[cid, pl.ds(sid * orows, orows)])


def _run_sc(tab, src, dst, zeros, nchunks, acc_rows, width):
    kern = functools.partial(
        pl.kernel,
        out_type=jax.ShapeDtypeStruct((_NC, _N3, width), jnp.bfloat16),
        mesh=_mesh(),
        scratch_types=[
            pltpu.VMEM((nchunks, _CH), jnp.int32),
            pltpu.VMEM((nchunks, _CH), jnp.int32),
            pltpu.VMEM((_NB, _CH, width), jnp.bfloat16),
            pltpu.VMEM_SHARED((acc_rows, width), jnp.bfloat16),
            pltpu.SemaphoreType.DMA((_NB,)),
            pltpu.SemaphoreType.DMA((_NB,)),
        ],
        compiler_params=pltpu.CompilerParams(use_tc_tiling_on_sc=False),
    )(functools.partial(_sc_body, nchunks, acc_rows, width))
    return kern(tab, src, dst, zeros)


_CAP = 163840 // _NW
_PK = 16384
_ST = 8


def _sc1_body(tab_hbm, pack_hbm, zsrc_hbm, d2d_hbm, zeros_hbm,
              out_hbm, pinp, csrc2d, cdst2d, rows_v, acc_sh, gsem, ssem):
    cid = lax.axis_index("c")
    sid = lax.axis_index("s")
    wid = sid * _NC + cid
    zrows = _ACC1_ROWS // _NS
    orows = _N3 // _NS
    pltpu.sync_copy(zeros_hbm, acc_sh.at[pl.ds(sid * zrows, zrows)])
    pltpu.sync_copy(pack_hbm.at[wid], pinp)
    pltpu.sync_copy(zsrc_hbm, csrc2d)
    pltpu.sync_copy(d2d_hbm, cdst2d)

    def _p1(i, off):
        p = pinp[pl.ds(i * 16, 16)]
        m = p < (_N3 * _PK)
        mi = m.astype(jnp.int32)
        cs = plsc.cumsum(mi)
        pos = off + cs - 1
        rowv = lax.shift_right_logical(pos, 7)
        colv = jnp.bitwise_and(pos, _CH - 1)
        plsc.store_scatter(csrc2d, [rowv, colv], jnp.bitwise_and(p, _PK - 1),
                           mask=m)
        plsc.store_scatter(cdst2d, [rowv, colv], lax.shift_right_logical(p, 14),
                           mask=m)
        return off + jnp.sum(mi)

    off = lax.fori_loop(0, _CAP // 16, _p1, jnp.int32(0))
    plsc.subcore_barrier()

    @pl.loop(0, (off + _CH - 1) // _CH)
    def _p3(c):
        pltpu.sync_copy(tab_hbm.at[csrc2d.at[c]], rows_v.at[0])
        pltpu.sync_copy(rows_v.at[0], acc_sh.at[cdst2d.at[c]], add=True)
    plsc.subcore_barrier()
    pltpu.sync_copy(acc_sh.at[pl.ds(sid * orows, orows)],
                    out_hbm.at[cid, pl.ds(sid * orows, orows)])


def _run_sc1(tab, packed, zsrc, d2d, zeros):
    kern = functools.partial(
        pl.kernel,
        out_type=jax.ShapeDtypeStruct((_NC, _N3, _W1), jnp.bfloat16),
        mesh=_mesh(),
        scratch_types=[
            pltpu.VMEM((_CAP,), jnp.int32),
            pltpu.VMEM((_CAP // _CH, _CH), jnp.int32),
            pltpu.VMEM((_CAP // _CH, _CH), jnp.int32),
            pltpu.VMEM((_NB, _CH, _W1), jnp.bfloat16),
            pltpu.VMEM_SHARED((_ACC1_ROWS, _W1), jnp.bfloat16),
            pltpu.SemaphoreType.DMA((_NB,)),
            pltpu.SemaphoreType.DMA((_NB,)),
        ],
        compiler_params=pltpu.CompilerParams(use_tc_tiling_on_sc=False,
                                             needs_layout_passes=False),
    )(_sc1_body)
    return kern(tab, packed, zsrc, d2d, zeros)


def _c_body(p_ref, x2_ref, w1r_ref, b1_ref, w2le_ref, c2_ref, w2re_ref, b2_ref,
            y2_ref, z2_ref):
    s = (p_ref[0].astype(jnp.float32) + p_ref[1].astype(jnp.float32))
    cnt = jnp.maximum(s[:, _D:_D + 1], 1.0)
    mean = s[:, :_D] / cnt
    h = mean + jnp.dot(x2_ref[...], w1r_ref[...],
                       preferred_element_type=jnp.float32) + b1_ref[...]
    h = jnp.maximum(h, 0.0)
    y2_ref[...] = (jnp.dot(h, w2le_ref[...],
                           preferred_element_type=jnp.float32)
                   + c2_ref[...]).astype(jnp.bfloat16)
    z2_ref[...] = jnp.dot(h, w2re_ref[...],
                          preferred_element_type=jnp.float32) + b2_ref[...]


def _run_c(p1, x2, w1r, b1, w2l_ext, c2_row, w2r_ext, b2_ext):
    return pl.pallas_call(
        _c_body,
        out_shape=(jax.ShapeDtypeStruct((_N3, _W2), jnp.bfloat16),
                   jax.ShapeDtypeStruct((_N3, _W2), jnp.float32)),
    )(p1, x2, w1r, b1, w2l_ext, c2_row, w2r_ext, b2_ext)


def _e_body(p_ref, z2_ref, out_ref):
    s = (p_ref[0].astype(jnp.float32) + p_ref[1].astype(jnp.float32))
    cnt = jnp.maximum(s[:, 48:49], 1.0)
    logits = s / cnt + z2_ref[...]
    col = lax.broadcasted_iota(jnp.int32, (_N3, _W2), 1)
    ml = jnp.where(col < _DO, logits, -1e30)
    m = jnp.max(ml, axis=1, keepdims=True)
    lse = jnp.log(jnp.sum(jnp.exp(ml - m), axis=1, keepdims=True)) + m
    out_ref[...] = (ml - lse)[:, :_DO]


def _run_e(p2, z2b):
    return pl.pallas_call(
        _e_body,
        out_shape=jax.ShapeDtypeStruct((_N3, _DO), jnp.float32),
    )(p2, z2b)


def kernel(x, edge_index1, edge_index2, W1l, b1, W1r, W2l, b2, W2r):
    f32 = jnp.float32
    x10k = x[:_N2]
    x2 = x[:_N3]

    w1l_ext = jnp.zeros((_D, _W1), f32).at[:, :_D].set(W1l)
    cnt_row = jnp.zeros((1, _W1), f32).at[0, _D].set(1.0)
    w2l_ext = jnp.zeros((_D, _W2), f32).at[:, :_DO].set(W2l)
    c2_row = jnp.zeros((1, _W2), f32).at[0, 48].set(1.0)
    w2r_ext = jnp.zeros((_D, _W2), f32).at[:, :_DO].set(W2r)
    b2_ext = jnp.zeros((1, _W2), f32).at[0, :_DO].set(b2)
    b1r = b1.reshape(1, _D)

    e1p = _NW * _CAP
    src1 = edge_index1[0].astype(jnp.int32)
    dst1 = edge_index1[1].astype(jnp.int32)
    pad = e1p - src1.shape[0]
    packed = dst1 * _PK + src1
    packed = jnp.concatenate([packed, jnp.full((pad,), _N2 * _PK, jnp.int32)])
    packed = packed.reshape(_NW, _CAP)
    zsrc = jnp.zeros((_CAP // _CH, _CH), jnp.int32)
    d2d = (_N3 + (jnp.arange(_CAP, dtype=jnp.int32) & 127)).reshape(
        _CAP // _CH, _CH)
    src2 = edge_index2[0].astype(jnp.int32).reshape(_NW, _C2, _CH)
    dst2 = edge_index2[1].astype(jnp.int32).reshape(_NW, _C2, _CH)

    zeros1 = jnp.zeros((_ACC1_ROWS // _NS, _W1), jnp.bfloat16)
    zeros2 = jnp.zeros((_N3 // _NS, _W2), jnp.bfloat16)

    y1ext = _run_a(x10k, w1l_ext, cnt_row)
    p1 = _run_sc1(y1ext, packed, zsrc, d2d, zeros1)
    y2ext, z2b = _run_c(p1, x2, W1r, b1r, w2l_ext, c2_row, w2r_ext, b2_ext)
    p2 = _run_sc(y2ext, src2, dst2, zeros2, _C2, _N3, _W2)
    return _run_e(p2, z2b)
